# baseline (speedup 1.0000x reference)
import functools

import jax
import jax.numpy as jnp
from jax import lax
from jax.experimental import pallas as pl
from jax.experimental.pallas import tpu as pltpu
from jax.experimental.pallas import tpu_sc as plsc

TOKEN_INDEX = 0
B, S, D = 4, 2048, 1024

_mesh = plsc.VectorSubcoreMesh(core_axis_name="c", subcore_axis_name="s")


@functools.partial(
    pl.kernel,
    mesh=_mesh,
    out_type=jax.ShapeDtypeStruct((B, D), jnp.float32),
    scratch_types=[pltpu.VMEM((D,), jnp.float32)],
)
def _extract(inp_hbm, out_hbm, row_v):
    cid = lax.axis_index("c")
    sid = lax.axis_index("s")
    wid = sid * 2 + cid

    @pl.when(wid < B)
    def _():
        pltpu.sync_copy(inp_hbm.at[wid, TOKEN_INDEX], row_v)
        pltpu.sync_copy(row_v, out_hbm.at[wid])


def kernel(inputs):
    return _extract(inputs)
